# baseline (speedup 1.0000x reference)
import functools

import jax
import jax.numpy as jnp
from jax import lax
from jax.experimental import pallas as pl
from jax.experimental.pallas import tpu as pltpu
from jax.experimental.pallas import tpu_sc as plsc

B, S, D = 4, 4096, 1024
NC, NS = 2, 16
NW = NC * NS
ROWS_W = S // NW
C = 32
NCHUNK = ROWS_W // C
CHUNK = C * D
VECS = CHUNK // 16


def _sc_body(x_hbm, pos_hbm, out_hbm, pos_v, x_v):
    w = lax.axis_index("s") * NC + lax.axis_index("c")
    for c in range(NCHUNK):
        s_off = (w * ROWS_W + c * C) * D
        pltpu.sync_copy(pos_hbm.at[pl.ds(s_off, CHUNK)], pos_v)
        for b in range(B):
            x_off = b * S * D + s_off
            pltpu.sync_copy(x_hbm.at[pl.ds(x_off, CHUNK)], x_v)

            def add16(i, _):
                sl = pl.ds(i * 16, 16)
                x_v[sl] = x_v[sl] + pos_v[sl]
                return _

            lax.fori_loop(0, VECS, add16, None)
            pltpu.sync_copy(x_v, out_hbm.at[pl.ds(x_off, CHUNK)])


def kernel(x, pos_table):
    xf = x.reshape(-1)
    pf = pos_table.reshape(-1)
    out = pl.kernel(
        _sc_body,
        out_type=jax.ShapeDtypeStruct((B * S * D,), jnp.float32),
        mesh=plsc.VectorSubcoreMesh(core_axis_name="c", subcore_axis_name="s"),
        scratch_types=[
            pltpu.VMEM((CHUNK,), jnp.float32),
            pltpu.VMEM((CHUNK,), jnp.float32),
        ],
    )(xf, pf)
    return out.reshape(B, S, D)


# speedup vs baseline: 1.5921x; 1.5921x over previous
import jax
import jax.numpy as jnp
from jax import lax
from jax.experimental import pallas as pl
from jax.experimental.pallas import tpu as pltpu
from jax.experimental.pallas import tpu_sc as plsc

B, S, D = 4, 4096, 1024
NC, NS = 2, 16
NW = NC * NS
ROWS_W = S // NW
C = 8
NCHUNK = ROWS_W // C
CHUNK = C * D
VECS = CHUNK // 16
UNROLL = 4


def _sc_body(x_hbm, pos_hbm, out_hbm, xbuf, pbuf, xld, xst, pld):
    w = lax.axis_index("s") * NC + lax.axis_index("c")
    base = w * ROWS_W * D

    def start_pos(c):
        off = base + c * CHUNK
        pltpu.async_copy(pos_hbm.at[pl.ds(off, CHUNK)], pbuf.at[c % 2],
                         pld.at[c % 2])

    def start_xload(c, b):
        off = b * S * D + base + c * CHUNK
        slot = b * 2 + c % 2
        pltpu.async_copy(x_hbm.at[pl.ds(off, CHUNK)], xbuf.at[slot],
                         xld.at[slot])

    def start_store(c, b):
        off = b * S * D + base + c * CHUNK
        slot = b * 2 + c % 2
        pltpu.async_copy(xbuf.at[slot], out_hbm.at[pl.ds(off, CHUNK)],
                         xst.at[slot])

    start_pos(0)
    for b in range(B):
        start_xload(0, b)

    for c in range(NCHUNK):
        par = c % 2
        if c + 1 < NCHUNK:
            start_pos(c + 1)
            for b in range(B):
                slot = b * 2 + (c + 1) % 2
                if c >= 1:
                    pltpu.make_async_copy(
                        xbuf.at[slot],
                        out_hbm.at[pl.ds(b * S * D + base + (c - 1) * CHUNK,
                                         CHUNK)],
                        xst.at[slot]).wait()
                start_xload(c + 1, b)
        pltpu.make_async_copy(pos_hbm.at[pl.ds(base + c * CHUNK, CHUNK)],
                              pbuf.at[par], pld.at[par]).wait()
        for b in range(B):
            slot = b * 2 + par
            pltpu.make_async_copy(
                x_hbm.at[pl.ds(b * S * D + base + c * CHUNK, CHUNK)],
                xbuf.at[slot], xld.at[slot]).wait()

        @plsc.parallel_loop(0, VECS, unroll=UNROLL)
        def _add(i):
            sl = pl.ds(i * 16, 16)
            p = pbuf[par, sl]
            for b in range(B):
                slot = b * 2 + par
                xbuf[slot, sl] = xbuf[slot, sl] + p

        for b in range(B):
            start_store(c, b)

    for c in (NCHUNK - 2, NCHUNK - 1):
        for b in range(B):
            slot = b * 2 + c % 2
            pltpu.make_async_copy(
                xbuf.at[slot],
                out_hbm.at[pl.ds(b * S * D + base + c * CHUNK, CHUNK)],
                xst.at[slot]).wait()


def kernel(x, pos_table):
    xf = x.reshape(-1)
    pf = pos_table.reshape(-1)
    out = pl.kernel(
        _sc_body,
        out_type=jax.ShapeDtypeStruct((B * S * D,), jnp.float32),
        mesh=plsc.VectorSubcoreMesh(core_axis_name="c", subcore_axis_name="s"),
        scratch_types=[
            pltpu.VMEM((2 * B, CHUNK), jnp.float32),
            pltpu.VMEM((2, CHUNK), jnp.float32),
            pltpu.SemaphoreType.DMA((2 * B,)),
            pltpu.SemaphoreType.DMA((2 * B,)),
            pltpu.SemaphoreType.DMA((2,)),
        ],
    )(xf, pf)
    return out.reshape(B, S, D)


# speedup vs baseline: 4.8088x; 3.0205x over previous
import jax
import jax.numpy as jnp
from jax import lax
from jax.experimental import pallas as pl
from jax.experimental.pallas import tpu as pltpu
from jax.experimental.pallas import tpu_sc as plsc

B, S, D = 4, 4096, 1024
NC, NS = 2, 16
NW = NC * NS
ROWS_W = S // NW
C = 8
NCHUNK = ROWS_W // C
CHUNK = C * D
VECS = CHUNK // 16
VECS_PER_ROW = D // 16
UNROLL = 4


def _sc_body(x_hbm, pos_hbm, out_hbm, xbuf, pbuf, xld, xst, pld):
    w = lax.axis_index("s") * NC + lax.axis_index("c")
    base = w * ROWS_W

    def start_pos(c):
        pltpu.async_copy(pos_hbm.at[pl.ds(base + c * C, C), :],
                         pbuf.at[c % 2], pld.at[c % 2])

    def start_xload(c, b):
        slot = b * 2 + c % 2
        pltpu.async_copy(x_hbm.at[pl.ds(b * S + base + c * C, C), :],
                         xbuf.at[slot], xld.at[slot])

    def start_store(c, b):
        slot = b * 2 + c % 2
        pltpu.async_copy(xbuf.at[slot],
                         out_hbm.at[pl.ds(b * S + base + c * C, C), :],
                         xst.at[slot])

    def wait_store(c, b):
        slot = b * 2 + c % 2
        pltpu.make_async_copy(
            xbuf.at[slot],
            out_hbm.at[pl.ds(b * S + base + c * C, C), :],
            xst.at[slot]).wait()

    start_pos(0)
    for b in range(B):
        start_xload(0, b)

    for c in range(NCHUNK):
        par = c % 2
        if c + 1 < NCHUNK:
            start_pos(c + 1)
            for b in range(B):
                if c >= 1:
                    wait_store(c - 1, b)
                start_xload(c + 1, b)
        pltpu.make_async_copy(pos_hbm.at[pl.ds(base + c * C, C), :],
                              pbuf.at[par], pld.at[par]).wait()
        for b in range(B):
            slot = b * 2 + par
            pltpu.make_async_copy(
                x_hbm.at[pl.ds(b * S + base + c * C, C), :],
                xbuf.at[slot], xld.at[slot]).wait()

        @plsc.parallel_loop(0, VECS, unroll=UNROLL)
        def _add(i):
            r = i // VECS_PER_ROW
            col = (i % VECS_PER_ROW) * 16
            sl = pl.ds(col, 16)
            p = pbuf[par, r, sl]
            for b in range(B):
                slot = b * 2 + par
                xbuf[slot, r, sl] = xbuf[slot, r, sl] + p

        for b in range(B):
            start_store(c, b)

    for c in (NCHUNK - 2, NCHUNK - 1):
        for b in range(B):
            wait_store(c, b)


def kernel(x, pos_table):
    x2 = x.reshape(B * S, D)
    out = pl.kernel(
        _sc_body,
        out_type=jax.ShapeDtypeStruct((B * S, D), jnp.float32),
        mesh=plsc.VectorSubcoreMesh(core_axis_name="c", subcore_axis_name="s"),
        compiler_params=pltpu.CompilerParams(use_tc_tiling_on_sc=True),
        scratch_types=[
            pltpu.VMEM((2 * B, C, D), jnp.float32),
            pltpu.VMEM((2, C, D), jnp.float32),
            pltpu.SemaphoreType.DMA((2 * B,)),
            pltpu.SemaphoreType.DMA((2 * B,)),
            pltpu.SemaphoreType.DMA((2,)),
        ],
    )(x2, pos_table)
    return out.reshape(B, S, D)
